# Initial kernel scaffold; baseline (speedup 1.0000x reference)
import jax
import jax.numpy as jnp
from jax.experimental import pallas as pl


def kernel(x, edge_index, W_self1, W_neigh1, b1, W_self2, W_neigh2, b2):
    raise NotImplementedError("write your pallas kernel here")


# speedup vs baseline: 5.8916x; 5.8916x over previous
import jax
import jax.numpy as jnp
from jax import lax
from jax.experimental import pallas as pl
from jax.experimental.pallas import tpu as pltpu
from jax.experimental.pallas import tpu_sc as plsc

NC = 2
NS = 16
NW = NC * NS
C = 128


def _make_sc_agg(n, d, e, with_deg):
    assert e % C == 0
    n_chunks = e // C
    tr = (n // NS) & ~7
    tail_off = tr * NS
    tail = n - tail_off
    zrows = 208
    assert tr % zrows == 0 and 0 <= tail <= zrows and tail % 8 == 0
    agg_iters = -(-n_chunks // NW)
    deg_iters = -(-n_chunks // NS)

    if with_deg:
        out_type = (jax.ShapeDtypeStruct((NC, n, d), jnp.float32),
                    jax.ShapeDtypeStruct((NC, n), jnp.float32))
    else:
        out_type = jax.ShapeDtypeStruct((NC, n, d), jnp.float32)

    scratch = [
        pltpu.VMEM_SHARED((n, d), jnp.float32),
        pltpu.VMEM((C,), jnp.int32),
        pltpu.VMEM((C,), jnp.int32),
        pltpu.VMEM((C, d), jnp.float32),
        pltpu.VMEM((zrows, d), jnp.float32),
        pltpu.SemaphoreType.DMA,
    ]
    if with_deg:
        scratch += [
            pltpu.VMEM_SHARED((n,), jnp.float32),
            pltpu.VMEM((C,), jnp.float32),
            pltpu.VMEM((2000,), jnp.float32),
        ]

    mesh = plsc.VectorSubcoreMesh(core_axis_name="c", subcore_axis_name="s")

    def body(x_hbm, src_hbm, dst_hbm, *refs):
        if with_deg:
            (agg_hbm, deg_hbm, agg_sh, src_v, dst_v, rows_v, zero_v, sem,
             deg_sh, ones_v, zerod_v) = refs
        else:
            (agg_hbm, agg_sh, src_v, dst_v, rows_v, zero_v, sem) = refs
        cid = lax.axis_index("c")
        sid = lax.axis_index("s")
        wid = sid * NC + cid
        tbase = pl.multiple_of(sid * tr, 8)

        def zb(i, carry):
            zero_v[i // (d // 16), pl.ds((i % (d // 16)) * 16, 16)] = (
                jnp.zeros((16,), jnp.float32))
            return carry
        lax.fori_loop(0, zrows * d // 16, zb, 0)
        for j in range(tr // zrows):
            pltpu.sync_copy(zero_v,
                            agg_sh.at[pl.ds(pl.multiple_of(tbase + j * zrows, 8),
                                            zrows), :])
        if tail:
            @pl.when(sid == NS - 1)
            def _():
                pltpu.sync_copy(zero_v.at[pl.ds(0, tail), :],
                                agg_sh.at[pl.ds(tail_off, tail), :])

        if with_deg:
            for k in range(C // 16):
                ones_v[pl.ds(k * 16, 16)] = jnp.full((16,), 1.0, jnp.float32)

            @pl.when(sid == 0)
            def _():
                def zd(i, _):
                    zerod_v[pl.ds(i * 16, 16)] = jnp.zeros((16,), jnp.float32)
                    return _
                lax.fori_loop(0, 2000 // 16, zd, 0)
                for j in range(n // 2000):
                    pltpu.sync_copy(zerod_v, deg_sh.at[pl.ds(j * 2000, 2000)])

        plsc.subcore_barrier()

        def eb(j, carry):
            g = wid + NW * j

            @pl.when(g < n_chunks)
            def _():
                base = g * C
                pltpu.sync_copy(src_hbm.at[pl.ds(base, C)], src_v)
                pltpu.async_copy(x_hbm.at[src_v], rows_v, sem).wait()
                pltpu.sync_copy(dst_hbm.at[pl.ds(base, C)], dst_v)
                pltpu.sync_copy(rows_v, agg_sh.at[dst_v], add=True)
            return carry
        lax.fori_loop(0, agg_iters, eb, 0)

        if with_deg:
            def db(j, carry):
                g = sid + NS * j

                @pl.when(g < n_chunks)
                def _():
                    pltpu.sync_copy(dst_hbm.at[pl.ds(g * C, C)], dst_v)
                    pltpu.sync_copy(ones_v, deg_sh.at[dst_v], add=True)
                return carry
            lax.fori_loop(0, deg_iters, db, 0)

        plsc.subcore_barrier()

        pltpu.sync_copy(agg_sh.at[pl.ds(tbase, tr), :],
                        agg_hbm.at[cid, pl.ds(tbase, tr), :])
        if tail:
            @pl.when(sid == NS - 1)
            def _():
                pltpu.sync_copy(agg_sh.at[pl.ds(tail_off, tail), :],
                                agg_hbm.at[cid, pl.ds(tail_off, tail), :])
        if with_deg:
            @pl.when(sid == 0)
            def _():
                pltpu.sync_copy(deg_sh, deg_hbm.at[cid])

    return pl.kernel(body, out_type=out_type, mesh=mesh,
                     scratch_types=scratch)


def _make_tc_dense(n, d, r):
    assert n % r == 0

    def body(x_ref, agg_ref, deg_ref, ws_ref, wn_ref, b_ref, o_ref):
        agg = agg_ref[0] + agg_ref[1]
        rdeg = 1.0 / jnp.maximum(deg_ref[...], 1.0)
        acc = jnp.dot(x_ref[...], ws_ref[...],
                      preferred_element_type=jnp.float32)
        acc = acc + jnp.dot(agg * rdeg, wn_ref[...],
                            preferred_element_type=jnp.float32)
        o_ref[...] = jnp.maximum(acc + b_ref[...], 0.0)

    return pl.pallas_call(
        body,
        grid=(n // r,),
        in_specs=[
            pl.BlockSpec((r, d), lambda i: (i, 0)),
            pl.BlockSpec((NC, r, d), lambda i: (0, i, 0)),
            pl.BlockSpec((r, 1), lambda i: (i, 0)),
            pl.BlockSpec((d, d), lambda i: (0, 0)),
            pl.BlockSpec((d, d), lambda i: (0, 0)),
            pl.BlockSpec((1, d), lambda i: (0, 0)),
        ],
        out_specs=pl.BlockSpec((r, d), lambda i: (i, 0)),
        out_shape=jax.ShapeDtypeStruct((n, d), jnp.float32),
    )


def kernel(x, edge_index, W_self1, W_neigh1, b1, W_self2, W_neigh2, b2):
    n, d = x.shape
    e = edge_index.shape[1]
    src = edge_index[0]
    dst = edge_index[1]

    sc_agg_deg = _make_sc_agg(n, d, e, with_deg=True)
    sc_agg = _make_sc_agg(n, d, e, with_deg=False)
    tc_dense = _make_tc_dense(n, d, 1000)

    aggp1, degp = sc_agg_deg(x, src, dst)
    deg_col = degp[0].reshape(n, 1)
    h = tc_dense(x, aggp1, deg_col, W_self1, W_neigh1, b1.reshape(1, d))
    aggp2 = sc_agg(h, src, dst)
    return tc_dense(h, aggp2, deg_col, W_self2, W_neigh2, b2.reshape(1, d))


# speedup vs baseline: 11.8776x; 2.0160x over previous
import jax
import jax.numpy as jnp
from jax import lax
from jax.experimental import pallas as pl
from jax.experimental.pallas import tpu as pltpu
from jax.experimental.pallas import tpu_sc as plsc

NC = 2
NS = 16
NW = NC * NS
C = 128


def _make_sc_agg(n, d, e, with_deg):
    assert e % NW == 0
    epw = e // NW
    full = epw // C
    tail = epw - full * C
    assert full % 2 == 0 and tail % 16 == 0 and tail < C
    tr = (n // NS) & ~7
    rtail_off = tr * NS
    rtail = n - rtail_off
    zrows = 48
    assert tr % zrows == 0 and 0 <= rtail <= zrows and rtail % 8 == 0

    if with_deg:
        out_type = (jax.ShapeDtypeStruct((NC, n, d), jnp.float32),
                    jax.ShapeDtypeStruct((NC, n), jnp.float32))
    else:
        out_type = jax.ShapeDtypeStruct((NC, n, d), jnp.float32)

    scratch = [
        pltpu.VMEM_SHARED((n, d), jnp.float32),
        [pltpu.VMEM((C,), jnp.int32)] * 2,
        [pltpu.VMEM((C,), jnp.int32)] * 2,
        [pltpu.VMEM((C, d), jnp.float32)] * 2,
        pltpu.VMEM((zrows, d), jnp.float32),
        [pltpu.SemaphoreType.DMA] * 2,
        [pltpu.SemaphoreType.DMA] * 2,
    ]
    if with_deg:
        scratch += [
            pltpu.VMEM_SHARED((n,), jnp.float32),
            [pltpu.VMEM((C,), jnp.int32)] * 2,
            pltpu.VMEM((C,), jnp.float32),
            pltpu.VMEM((2000,), jnp.float32),
        ]
    if tail:
        scratch += [[pltpu.VMEM((tail,), jnp.int32)] * 3]

    mesh = plsc.VectorSubcoreMesh(core_axis_name="c", subcore_axis_name="s")

    def body(x_hbm, src_hbm, dst_hbm, *refs):
        if tail:
            refs, (tsx, tdx, tdx2) = refs[:-1], refs[-1]
        if with_deg:
            (agg_hbm, deg_hbm, agg_sh, sidx, didx, rows, zero_v, gsem, isem,
             deg_sh, didx2, ones_v, zerod_v) = refs
        else:
            (agg_hbm, agg_sh, sidx, didx, rows, zero_v, gsem, isem) = refs
        cid = lax.axis_index("c")
        sid = lax.axis_index("s")
        wid = sid * NC + cid
        tbase = pl.multiple_of(sid * tr, 8)
        ebase = pl.multiple_of(wid * epw, 8)

        def zb(i, carry):
            zero_v[i // (d // 16), pl.ds((i % (d // 16)) * 16, 16)] = (
                jnp.zeros((16,), jnp.float32))
            return carry
        lax.fori_loop(0, zrows * d // 16, zb, 0)
        for j in range(tr // zrows):
            pltpu.sync_copy(zero_v,
                            agg_sh.at[pl.ds(pl.multiple_of(tbase + j * zrows, 8),
                                            zrows), :])
        if rtail:
            @pl.when(sid == NS - 1)
            def _():
                pltpu.sync_copy(zero_v.at[pl.ds(0, rtail), :],
                                agg_sh.at[pl.ds(rtail_off, rtail), :])

        if with_deg:
            for k in range(C // 16):
                ones_v[pl.ds(k * 16, 16)] = jnp.full((16,), 1.0, jnp.float32)
            pbase = pl.multiple_of((sid * NC + (1 - cid)) * epw, 8)

            @pl.when(sid == 0)
            def _():
                def zd(i, carry):
                    zerod_v[pl.ds(i * 16, 16)] = jnp.zeros((16,), jnp.float32)
                    return carry
                lax.fori_loop(0, 2000 // 16, zd, 0)
                for j in range(n // 2000):
                    pltpu.sync_copy(zerod_v, deg_sh.at[pl.ds(j * 2000, 2000)])

        plsc.subcore_barrier()

        def load_idx(j, b, nidx):
            off = pl.multiple_of(ebase + j * C, 8)
            pltpu.async_copy(src_hbm.at[pl.ds(off, nidx)],
                             sidx[b].at[pl.ds(0, nidx)], isem[b])
            pltpu.async_copy(dst_hbm.at[pl.ds(off, nidx)],
                             didx[b].at[pl.ds(0, nidx)], isem[b])
            if with_deg:
                poff = pl.multiple_of(pbase + j * C, 8)
                pltpu.async_copy(dst_hbm.at[pl.ds(poff, nidx)],
                                 didx2[b].at[pl.ds(0, nidx)], isem[b])

        def wait_idx(b, nidx):
            ncopies = 3 if with_deg else 2
            for _ in range(ncopies):
                pltpu.make_async_copy(src_hbm.at[pl.ds(0, nidx)],
                                      sidx[b].at[pl.ds(0, nidx)],
                                      isem[b]).wait()

        def start_gather(b):
            return pltpu.async_copy(x_hbm.at[sidx[b]], rows[b], gsem[b])

        def wait_gather(b):
            pltpu.make_async_copy(x_hbm.at[pl.ds(0, C), :], rows[b],
                                  gsem[b]).wait()

        def scatter(b):
            pltpu.sync_copy(rows[b], agg_sh.at[didx[b]], add=True)
            if with_deg:
                pltpu.sync_copy(ones_v, deg_sh.at[didx[b]], add=True)
                pltpu.sync_copy(ones_v, deg_sh.at[didx2[b]], add=True)

        load_idx(0, 0, C)
        wait_idx(0, C)
        start_gather(0)

        def eb(jj, carry):
            for b in range(2):
                j = jj * 2 + b

                @pl.when(j + 1 < full)
                def _():
                    load_idx(j + 1, 1 - b, C)
                wait_gather(b)

                @pl.when(j + 1 < full)
                def _():
                    wait_idx(1 - b, C)
                    start_gather(1 - b)
                scatter(b)
            return carry
        lax.fori_loop(0, (full + 1) // 2, eb, 0)

        if tail:
            toff = pl.multiple_of(ebase + full * C, 8)
            pltpu.async_copy(src_hbm.at[pl.ds(toff, tail)], tsx, isem[0])
            pltpu.async_copy(dst_hbm.at[pl.ds(toff, tail)], tdx, isem[0])
            if with_deg:
                ptoff = pl.multiple_of(pbase + full * C, 8)
                pltpu.async_copy(dst_hbm.at[pl.ds(ptoff, tail)], tdx2,
                                 isem[0])
            for _ in range(3 if with_deg else 2):
                pltpu.make_async_copy(src_hbm.at[pl.ds(0, tail)], tsx,
                                      isem[0]).wait()
            pltpu.async_copy(x_hbm.at[tsx],
                             rows[0].at[pl.ds(0, tail), :], gsem[0]).wait()
            pltpu.sync_copy(rows[0].at[pl.ds(0, tail), :],
                            agg_sh.at[tdx], add=True)
            if with_deg:
                ones_t = ones_v.at[pl.ds(0, tail)]
                pltpu.sync_copy(ones_t, deg_sh.at[tdx], add=True)
                pltpu.sync_copy(ones_t, deg_sh.at[tdx2], add=True)

        plsc.subcore_barrier()

        pltpu.sync_copy(agg_sh.at[pl.ds(tbase, tr), :],
                        agg_hbm.at[cid, pl.ds(tbase, tr), :])
        if rtail:
            @pl.when(sid == NS - 1)
            def _():
                pltpu.sync_copy(agg_sh.at[pl.ds(rtail_off, rtail), :],
                                agg_hbm.at[cid, pl.ds(rtail_off, rtail), :])
        if with_deg:
            @pl.when(sid == 0)
            def _():
                pltpu.sync_copy(deg_sh, deg_hbm.at[cid])

    return pl.kernel(body, out_type=out_type, mesh=mesh,
                     scratch_types=scratch)


def _make_tc_dense(n, d, r):
    assert n % r == 0

    def body(x_ref, agg_ref, deg_ref, ws_ref, wn_ref, b_ref, o_ref):
        agg = agg_ref[0] + agg_ref[1]
        rdeg = 1.0 / jnp.maximum(deg_ref[...], 1.0)
        acc = jnp.dot(x_ref[...], ws_ref[...],
                      preferred_element_type=jnp.float32)
        acc = acc + jnp.dot(agg * rdeg, wn_ref[...],
                            preferred_element_type=jnp.float32)
        o_ref[...] = jnp.maximum(acc + b_ref[...], 0.0)

    return pl.pallas_call(
        body,
        grid=(n // r,),
        in_specs=[
            pl.BlockSpec((r, d), lambda i: (i, 0)),
            pl.BlockSpec((NC, r, d), lambda i: (0, i, 0)),
            pl.BlockSpec((r, 1), lambda i: (i, 0)),
            pl.BlockSpec((d, d), lambda i: (0, 0)),
            pl.BlockSpec((d, d), lambda i: (0, 0)),
            pl.BlockSpec((1, d), lambda i: (0, 0)),
        ],
        out_specs=pl.BlockSpec((r, d), lambda i: (i, 0)),
        out_shape=jax.ShapeDtypeStruct((n, d), jnp.float32),
    )


def kernel(x, edge_index, W_self1, W_neigh1, b1, W_self2, W_neigh2, b2):
    n, d = x.shape
    e = edge_index.shape[1]
    src = edge_index[0]
    dst = edge_index[1]

    sc_agg_deg = _make_sc_agg(n, d, e, with_deg=True)
    sc_agg = _make_sc_agg(n, d, e, with_deg=False)
    tc_dense = _make_tc_dense(n, d, 1000)

    aggp1, degp = sc_agg_deg(x, src, dst)
    deg_col = degp[0].reshape(n, 1)
    h = tc_dense(x, aggp1, deg_col, W_self1, W_neigh1, b1.reshape(1, d))
    aggp2 = sc_agg(h, src, dst)
    return tc_dense(h, aggp2, deg_col, W_self2, W_neigh2, b2.reshape(1, d))


# speedup vs baseline: 11.9765x; 1.0083x over previous
import jax
import jax.numpy as jnp
from jax import lax
from jax.experimental import pallas as pl
from jax.experimental.pallas import tpu as pltpu
from jax.experimental.pallas import tpu_sc as plsc

NC = 2
NS = 16
NW = NC * NS
C = 128


def _make_sc_agg(n, d, e, with_deg):
    assert e % NW == 0
    epw = e // NW
    full = epw // C
    tail = epw - full * C
    assert full % 6 == 0 and tail % 16 == 0 and tail < C
    tr = (n // NS) & ~7
    rtail_off = tr * NS
    rtail = n - rtail_off
    zrows = 48
    assert tr % zrows == 0 and 0 <= rtail <= zrows and rtail % 8 == 0

    if with_deg:
        out_type = (jax.ShapeDtypeStruct((NC, n, d), jnp.float32),
                    jax.ShapeDtypeStruct((NC, n), jnp.float32))
    else:
        out_type = jax.ShapeDtypeStruct((NC, n, d), jnp.float32)

    scratch = [
        pltpu.VMEM_SHARED((n, d), jnp.float32),
        [pltpu.VMEM((C,), jnp.int32)] * 3,
        [pltpu.VMEM((C,), jnp.int32)] * 3,
        [pltpu.VMEM((C, d), jnp.float32)] * 2,
        pltpu.VMEM((zrows, d), jnp.float32),
        [pltpu.SemaphoreType.DMA] * 2,
        [pltpu.SemaphoreType.DMA] * 3,
        [pltpu.SemaphoreType.DMA] * 2,
    ]
    if with_deg:
        scratch += [
            pltpu.VMEM_SHARED((n,), jnp.float32),
            [pltpu.VMEM((C,), jnp.int32)] * 3,
            pltpu.VMEM((C,), jnp.float32),
            pltpu.VMEM((2000,), jnp.float32),
        ]
    if tail:
        scratch += [[pltpu.VMEM((tail,), jnp.int32)] * 3]

    mesh = plsc.VectorSubcoreMesh(core_axis_name="c", subcore_axis_name="s")

    def body(x_hbm, src_hbm, dst_hbm, *refs):
        if tail:
            refs, (tsx, tdx, tdx2) = refs[:-1], refs[-1]
        if with_deg:
            (agg_hbm, deg_hbm, agg_sh, sidx, didx, rows, zero_v, gsem, isem,
             ssem, deg_sh, didx2, ones_v, zerod_v) = refs
        else:
            (agg_hbm, agg_sh, sidx, didx, rows, zero_v, gsem, isem,
             ssem) = refs
        cid = lax.axis_index("c")
        sid = lax.axis_index("s")
        wid = sid * NC + cid
        tbase = pl.multiple_of(sid * tr, 8)
        ebase = pl.multiple_of(wid * epw, 8)

        def zb(i, carry):
            zero_v[i // (d // 16), pl.ds((i % (d // 16)) * 16, 16)] = (
                jnp.zeros((16,), jnp.float32))
            return carry
        lax.fori_loop(0, zrows * d // 16, zb, 0)
        for j in range(tr // zrows):
            pltpu.sync_copy(zero_v,
                            agg_sh.at[pl.ds(pl.multiple_of(tbase + j * zrows, 8),
                                            zrows), :])
        if rtail:
            @pl.when(sid == NS - 1)
            def _():
                pltpu.sync_copy(zero_v.at[pl.ds(0, rtail), :],
                                agg_sh.at[pl.ds(rtail_off, rtail), :])

        if with_deg:
            for k in range(C // 16):
                ones_v[pl.ds(k * 16, 16)] = jnp.full((16,), 1.0, jnp.float32)
            pbase = pl.multiple_of((sid * NC + (1 - cid)) * epw, 8)

            @pl.when(sid == 0)
            def _():
                def zd(i, carry):
                    zerod_v[pl.ds(i * 16, 16)] = jnp.zeros((16,), jnp.float32)
                    return carry
                lax.fori_loop(0, 2000 // 16, zd, 0)
                for j in range(n // 2000):
                    pltpu.sync_copy(zerod_v, deg_sh.at[pl.ds(j * 2000, 2000)])

        plsc.subcore_barrier()

        def load_idx(j, q):
            off = pl.multiple_of(ebase + j * C, 8)
            pltpu.async_copy(src_hbm.at[pl.ds(off, C)], sidx[q], isem[q])
            pltpu.async_copy(dst_hbm.at[pl.ds(off, C)], didx[q], isem[q])
            if with_deg:
                poff = pl.multiple_of(pbase + j * C, 8)
                pltpu.async_copy(dst_hbm.at[pl.ds(poff, C)], didx2[q],
                                 isem[q])

        def wait_idx(q):
            for _ in range(3 if with_deg else 2):
                pltpu.make_async_copy(src_hbm.at[pl.ds(0, C)], sidx[q],
                                      isem[q]).wait()

        def start_gather(q, b):
            pltpu.async_copy(x_hbm.at[sidx[q]], rows[b], gsem[b])

        def wait_gather(b):
            pltpu.make_async_copy(x_hbm.at[pl.ds(0, C), :], rows[b],
                                  gsem[b]).wait()

        def issue_scatters(q, b):
            pltpu.async_copy(rows[b], agg_sh.at[didx[q]], ssem[b], add=True)
            if with_deg:
                pltpu.async_copy(ones_v, deg_sh.at[didx[q]], ssem[b],
                                 add=True)
                pltpu.async_copy(ones_v, deg_sh.at[didx2[q]], ssem[b],
                                 add=True)

        def drain_scatters(b):
            pltpu.make_async_copy(rows[b], agg_sh.at[pl.ds(0, C), :],
                                  ssem[b]).wait()
            if with_deg:
                for _ in range(2):
                    pltpu.make_async_copy(ones_v, deg_sh.at[pl.ds(0, C)],
                                          ssem[b]).wait()

        load_idx(0, 0)
        wait_idx(0)
        start_gather(0, 0)

        def eb(jj, carry):
            for u in range(6):
                j = jj * 6 + u
                b = u % 2
                q = u % 3
                q1 = (u + 1) % 3

                @pl.when(j + 1 < full)
                def _():
                    load_idx(j + 1, q1)
                wait_gather(b)

                @pl.when(j > 0)
                def _():
                    drain_scatters(1 - b)

                @pl.when(j + 1 < full)
                def _():
                    wait_idx(q1)
                    start_gather(q1, 1 - b)
                issue_scatters(q, b)
            return carry
        lax.fori_loop(0, full // 6, eb, 0)
        drain_scatters((full - 1) % 2)

        if tail:
            toff = pl.multiple_of(ebase + full * C, 8)
            pltpu.async_copy(src_hbm.at[pl.ds(toff, tail)], tsx, isem[0])
            pltpu.async_copy(dst_hbm.at[pl.ds(toff, tail)], tdx, isem[0])
            if with_deg:
                ptoff = pl.multiple_of(pbase + full * C, 8)
                pltpu.async_copy(dst_hbm.at[pl.ds(ptoff, tail)], tdx2,
                                 isem[0])
            for _ in range(3 if with_deg else 2):
                pltpu.make_async_copy(src_hbm.at[pl.ds(0, tail)], tsx,
                                      isem[0]).wait()
            pltpu.async_copy(x_hbm.at[tsx],
                             rows[0].at[pl.ds(0, tail), :], gsem[0]).wait()
            pltpu.sync_copy(rows[0].at[pl.ds(0, tail), :],
                            agg_sh.at[tdx], add=True)
            if with_deg:
                ones_t = ones_v.at[pl.ds(0, tail)]
                pltpu.sync_copy(ones_t, deg_sh.at[tdx], add=True)
                pltpu.sync_copy(ones_t, deg_sh.at[tdx2], add=True)

        plsc.subcore_barrier()

        pltpu.sync_copy(agg_sh.at[pl.ds(tbase, tr), :],
                        agg_hbm.at[cid, pl.ds(tbase, tr), :])
        if rtail:
            @pl.when(sid == NS - 1)
            def _():
                pltpu.sync_copy(agg_sh.at[pl.ds(rtail_off, rtail), :],
                                agg_hbm.at[cid, pl.ds(rtail_off, rtail), :])
        if with_deg:
            @pl.when(sid == 0)
            def _():
                pltpu.sync_copy(deg_sh, deg_hbm.at[cid])

    return pl.kernel(body, out_type=out_type, mesh=mesh,
                     scratch_types=scratch)


def _make_tc_dense(n, d, r):
    assert n % r == 0

    def body(x_ref, agg_ref, deg_ref, ws_ref, wn_ref, b_ref, o_ref):
        agg = agg_ref[0] + agg_ref[1]
        rdeg = 1.0 / jnp.maximum(deg_ref[...], 1.0)
        acc = jnp.dot(x_ref[...], ws_ref[...],
                      preferred_element_type=jnp.float32)
        acc = acc + jnp.dot(agg * rdeg, wn_ref[...],
                            preferred_element_type=jnp.float32)
        o_ref[...] = jnp.maximum(acc + b_ref[...], 0.0)

    return pl.pallas_call(
        body,
        grid=(n // r,),
        in_specs=[
            pl.BlockSpec((r, d), lambda i: (i, 0)),
            pl.BlockSpec((NC, r, d), lambda i: (0, i, 0)),
            pl.BlockSpec((r, 1), lambda i: (i, 0)),
            pl.BlockSpec((d, d), lambda i: (0, 0)),
            pl.BlockSpec((d, d), lambda i: (0, 0)),
            pl.BlockSpec((1, d), lambda i: (0, 0)),
        ],
        out_specs=pl.BlockSpec((r, d), lambda i: (i, 0)),
        out_shape=jax.ShapeDtypeStruct((n, d), jnp.float32),
    )


def kernel(x, edge_index, W_self1, W_neigh1, b1, W_self2, W_neigh2, b2):
    n, d = x.shape
    e = edge_index.shape[1]
    src = edge_index[0]
    dst = edge_index[1]

    sc_agg_deg = _make_sc_agg(n, d, e, with_deg=True)
    sc_agg = _make_sc_agg(n, d, e, with_deg=False)
    tc_dense = _make_tc_dense(n, d, 1000)

    aggp1, degp = sc_agg_deg(x, src, dst)
    deg_col = degp[0].reshape(n, 1)
    h = tc_dense(x, aggp1, deg_col, W_self1, W_neigh1, b1.reshape(1, d))
    aggp2 = sc_agg(h, src, dst)
    return tc_dense(h, aggp2, deg_col, W_self2, W_neigh2, b2.reshape(1, d))


# speedup vs baseline: 13.7650x; 1.1493x over previous
import jax
import jax.numpy as jnp
from jax import lax
from jax.experimental import pallas as pl
from jax.experimental.pallas import tpu as pltpu
from jax.experimental.pallas import tpu_sc as plsc

NC = 2
NS = 16
NW = NC * NS
C = 128


def _make_sc_agg(n, d, e, with_deg):
    assert e % NW == 0
    epw = e // NW
    full = epw // C
    tail = epw - full * C
    assert full % 6 == 0 and tail % 16 == 0 and tail < C
    tr = (n // NS) & ~7
    rtail_off = tr * NS
    rtail = n - rtail_off
    zrows = 48
    assert tr % zrows == 0 and 0 <= rtail <= zrows and rtail % 8 == 0

    if with_deg:
        out_type = (jax.ShapeDtypeStruct((NC, n, d), jnp.float32),
                    jax.ShapeDtypeStruct((NC, n), jnp.float32))
    else:
        out_type = jax.ShapeDtypeStruct((NC, n, d), jnp.float32)

    scratch = [
        pltpu.VMEM_SHARED((n, d), jnp.float32),
        [pltpu.VMEM((C,), jnp.int32)] * 3,
        [pltpu.VMEM((C,), jnp.int32)] * 3,
        [pltpu.VMEM((C, d), jnp.float32)] * 2,
        pltpu.VMEM((zrows, d), jnp.float32),
        [pltpu.SemaphoreType.DMA] * 2,
        [pltpu.SemaphoreType.DMA] * 3,
        [pltpu.SemaphoreType.DMA] * 2,
    ]
    if with_deg:
        scratch += [
            pltpu.VMEM_SHARED((n,), jnp.float32),
            [pltpu.VMEM((C,), jnp.int32)] * 3,
            pltpu.VMEM((C,), jnp.float32),
            pltpu.VMEM((2000,), jnp.float32),
        ]
    if tail:
        scratch += [[pltpu.VMEM((tail,), jnp.int32)] * 3]

    mesh = plsc.VectorSubcoreMesh(core_axis_name="c", subcore_axis_name="s")

    def body(x_hbm, src_hbm, dst_hbm, *refs):
        if tail:
            refs, (tsx, tdx, tdx2) = refs[:-1], refs[-1]
        if with_deg:
            (agg_hbm, deg_hbm, agg_sh, sidx, didx, rows, zero_v, gsem, isem,
             ssem, deg_sh, didx2, ones_v, zerod_v) = refs
        else:
            (agg_hbm, agg_sh, sidx, didx, rows, zero_v, gsem, isem,
             ssem) = refs
        cid = lax.axis_index("c")
        sid = lax.axis_index("s")
        wid = sid * NC + cid
        tbase = pl.multiple_of(sid * tr, 8)
        ebase = pl.multiple_of(wid * epw, 8)

        def zb(i, carry):
            zero_v[i // (d // 16), pl.ds((i % (d // 16)) * 16, 16)] = (
                jnp.zeros((16,), jnp.float32))
            return carry
        lax.fori_loop(0, zrows * d // 16, zb, 0)
        for j in range(tr // zrows):
            pltpu.sync_copy(zero_v,
                            agg_sh.at[pl.ds(pl.multiple_of(tbase + j * zrows, 8),
                                            zrows), :])
        if rtail:
            @pl.when(sid == NS - 1)
            def _():
                pltpu.sync_copy(zero_v.at[pl.ds(0, rtail), :],
                                agg_sh.at[pl.ds(rtail_off, rtail), :])

        if with_deg:
            for k in range(C // 16):
                ones_v[pl.ds(k * 16, 16)] = jnp.full((16,), 1.0, jnp.float32)
            pbase = pl.multiple_of((sid * NC + (1 - cid)) * epw, 8)

            @pl.when(sid == 0)
            def _():
                def zd(i, carry):
                    zerod_v[pl.ds(i * 16, 16)] = jnp.zeros((16,), jnp.float32)
                    return carry
                lax.fori_loop(0, 2000 // 16, zd, 0)
                for j in range(n // 2000):
                    pltpu.sync_copy(zerod_v, deg_sh.at[pl.ds(j * 2000, 2000)])

        plsc.subcore_barrier()

        def load_idx(j, q):
            off = pl.multiple_of(ebase + j * C, 8)
            pltpu.async_copy(src_hbm.at[pl.ds(off, C)], sidx[q], isem[q])
            pltpu.async_copy(dst_hbm.at[pl.ds(off, C)], didx[q], isem[q])
            if with_deg:
                poff = pl.multiple_of(pbase + j * C, 8)
                pltpu.async_copy(dst_hbm.at[pl.ds(poff, C)], didx2[q],
                                 isem[q])

        def wait_idx(q):
            for _ in range(3 if with_deg else 2):
                pltpu.make_async_copy(src_hbm.at[pl.ds(0, C)], sidx[q],
                                      isem[q]).wait()

        def start_gather(q, b):
            pltpu.async_copy(x_hbm.at[sidx[q]], rows[b], gsem[b])

        def wait_gather(b):
            pltpu.make_async_copy(x_hbm.at[pl.ds(0, C), :], rows[b],
                                  gsem[b]).wait()

        def issue_scatters(q, b):
            pltpu.async_copy(rows[b], agg_sh.at[didx[q]], ssem[b], add=True)
            if with_deg:
                pltpu.async_copy(ones_v, deg_sh.at[didx[q]], ssem[b],
                                 add=True)
                pltpu.async_copy(ones_v, deg_sh.at[didx2[q]], ssem[b],
                                 add=True)

        def drain_scatters(b):
            pltpu.make_async_copy(rows[b], agg_sh.at[pl.ds(0, C), :],
                                  ssem[b]).wait()
            if with_deg:
                for _ in range(2):
                    pltpu.make_async_copy(ones_v, deg_sh.at[pl.ds(0, C)],
                                          ssem[b]).wait()

        load_idx(0, 0)
        wait_idx(0)
        start_gather(0, 0)
        load_idx(1, 1)

        def eb(jj, carry):
            for u in range(6):
                j = jj * 6 + u
                b = u % 2
                q = u % 3
                q1 = (u + 1) % 3
                q2 = (u + 2) % 3

                @pl.when(j > 0)
                def _():
                    drain_scatters(1 - b)

                @pl.when(j + 1 < full)
                def _():
                    wait_idx(q1)
                    start_gather(q1, 1 - b)
                wait_gather(b)

                @pl.when(j + 2 < full)
                def _():
                    load_idx(j + 2, q2)
                issue_scatters(q, b)
            return carry
        lax.fori_loop(0, full // 6, eb, 0)
        drain_scatters((full - 1) % 2)

        if tail:
            toff = pl.multiple_of(ebase + full * C, 8)
            pltpu.async_copy(src_hbm.at[pl.ds(toff, tail)], tsx, isem[0])
            pltpu.async_copy(dst_hbm.at[pl.ds(toff, tail)], tdx, isem[0])
            if with_deg:
                ptoff = pl.multiple_of(pbase + full * C, 8)
                pltpu.async_copy(dst_hbm.at[pl.ds(ptoff, tail)], tdx2,
                                 isem[0])
            for _ in range(3 if with_deg else 2):
                pltpu.make_async_copy(src_hbm.at[pl.ds(0, tail)], tsx,
                                      isem[0]).wait()
            pltpu.async_copy(x_hbm.at[tsx],
                             rows[0].at[pl.ds(0, tail), :], gsem[0]).wait()
            pltpu.sync_copy(rows[0].at[pl.ds(0, tail), :],
                            agg_sh.at[tdx], add=True)
            if with_deg:
                ones_t = ones_v.at[pl.ds(0, tail)]
                pltpu.sync_copy(ones_t, deg_sh.at[tdx], add=True)
                pltpu.sync_copy(ones_t, deg_sh.at[tdx2], add=True)

        plsc.subcore_barrier()

        pltpu.sync_copy(agg_sh.at[pl.ds(tbase, tr), :],
                        agg_hbm.at[cid, pl.ds(tbase, tr), :])
        if rtail:
            @pl.when(sid == NS - 1)
            def _():
                pltpu.sync_copy(agg_sh.at[pl.ds(rtail_off, rtail), :],
                                agg_hbm.at[cid, pl.ds(rtail_off, rtail), :])
        if with_deg:
            @pl.when(sid == 0)
            def _():
                pltpu.sync_copy(deg_sh, deg_hbm.at[cid])

    return pl.kernel(body, out_type=out_type, mesh=mesh,
                     scratch_types=scratch)


def _make_tc_dense(n, d, r):
    assert n % r == 0

    def body(x_ref, agg_ref, deg_ref, ws_ref, wn_ref, b_ref, o_ref):
        agg = agg_ref[0] + agg_ref[1]
        rdeg = 1.0 / jnp.maximum(deg_ref[...], 1.0)
        acc = jnp.dot(x_ref[...], ws_ref[...],
                      preferred_element_type=jnp.float32)
        acc = acc + jnp.dot(agg * rdeg, wn_ref[...],
                            preferred_element_type=jnp.float32)
        o_ref[...] = jnp.maximum(acc + b_ref[...], 0.0)

    return pl.pallas_call(
        body,
        grid=(n // r,),
        in_specs=[
            pl.BlockSpec((r, d), lambda i: (i, 0)),
            pl.BlockSpec((NC, r, d), lambda i: (0, i, 0)),
            pl.BlockSpec((r, 1), lambda i: (i, 0)),
            pl.BlockSpec((d, d), lambda i: (0, 0)),
            pl.BlockSpec((d, d), lambda i: (0, 0)),
            pl.BlockSpec((1, d), lambda i: (0, 0)),
        ],
        out_specs=pl.BlockSpec((r, d), lambda i: (i, 0)),
        out_shape=jax.ShapeDtypeStruct((n, d), jnp.float32),
    )


def kernel(x, edge_index, W_self1, W_neigh1, b1, W_self2, W_neigh2, b2):
    n, d = x.shape
    e = edge_index.shape[1]
    src = edge_index[0]
    dst = edge_index[1]

    sc_agg_deg = _make_sc_agg(n, d, e, with_deg=True)
    sc_agg = _make_sc_agg(n, d, e, with_deg=False)
    tc_dense = _make_tc_dense(n, d, 1000)

    aggp1, degp = sc_agg_deg(x, src, dst)
    deg_col = degp[0].reshape(n, 1)
    h = tc_dense(x, aggp1, deg_col, W_self1, W_neigh1, b1.reshape(1, d))
    aggp2 = sc_agg(h, src, dst)
    return tc_dense(h, aggp2, deg_col, W_self2, W_neigh2, b2.reshape(1, d))


# speedup vs baseline: 14.8604x; 1.0796x over previous
import jax
import jax.numpy as jnp
from jax import lax
from jax.experimental import pallas as pl
from jax.experimental.pallas import tpu as pltpu
from jax.experimental.pallas import tpu_sc as plsc

NC = 2
NS = 16
NW = NC * NS
C = 112
ND = 8


def _make_sc_agg(n, d, epw, with_deg):
    assert epw % C == 0
    full = epw // C
    assert full % 3 == 0
    tr = (n // NS) & ~7
    rtail_off = tr * NS
    rtail = n - rtail_off
    zrows = 24
    assert tr % zrows == 0 and 0 <= rtail <= zrows and rtail % 8 == 0

    if with_deg:
        out_type = (jax.ShapeDtypeStruct((NC, n, d), jnp.float32),
                    jax.ShapeDtypeStruct((NC, n + ND), jnp.float32))
    else:
        out_type = jax.ShapeDtypeStruct((NC, n, d), jnp.float32)

    scratch = [
        pltpu.VMEM_SHARED((n + ND, d), jnp.float32),
        [pltpu.VMEM((C,), jnp.int32)] * 3,
        [pltpu.VMEM((C,), jnp.int32)] * 3,
        [pltpu.VMEM((C, d), jnp.float32)] * 3,
        pltpu.VMEM((zrows, d), jnp.float32),
        [pltpu.SemaphoreType.DMA] * 3,
        [pltpu.SemaphoreType.DMA] * 3,
        [pltpu.SemaphoreType.DMA] * 3,
        [pltpu.SemaphoreType.DMA] * 3,
    ]
    if with_deg:
        scratch += [
            pltpu.VMEM_SHARED((n + ND,), jnp.float32),
            pltpu.VMEM((C,), jnp.float32),
            pltpu.VMEM((2000,), jnp.float32),
        ]

    mesh = plsc.VectorSubcoreMesh(core_axis_name="c", subcore_axis_name="s")

    def body(x_hbm, src_hbm, dst_hbm, *refs):
        if with_deg:
            (agg_hbm, deg_hbm, agg_sh, sidx, didx, rows, zero_v, gsem, isem,
             jsem, ssem, deg_sh, ones_v, zerod_v) = refs
        else:
            (agg_hbm, agg_sh, sidx, didx, rows, zero_v, gsem, isem, jsem,
             ssem) = refs
        cid = lax.axis_index("c")
        sid = lax.axis_index("s")
        wid = sid * NC + cid
        tbase = pl.multiple_of(sid * tr, 8)
        ebase = pl.multiple_of(wid * epw, 8)

        def zb(i, carry):
            zero_v[i // (d // 16), pl.ds((i % (d // 16)) * 16, 16)] = (
                jnp.zeros((16,), jnp.float32))
            return carry
        lax.fori_loop(0, zrows * d // 16, zb, 0)
        for j in range(tr // zrows):
            pltpu.sync_copy(zero_v,
                            agg_sh.at[pl.ds(pl.multiple_of(tbase + j * zrows, 8),
                                            zrows), :])
        if rtail:
            @pl.when(sid == NS - 1)
            def _():
                pltpu.sync_copy(zero_v.at[pl.ds(0, rtail), :],
                                agg_sh.at[pl.ds(rtail_off, rtail), :])

        if with_deg:
            for k in range(C // 16):
                ones_v[pl.ds(k * 16, 16)] = jnp.full((16,), 1.0, jnp.float32)

            @pl.when(sid == 0)
            def _():
                def zd(i, carry):
                    zerod_v[pl.ds(i * 16, 16)] = jnp.zeros((16,), jnp.float32)
                    return carry
                lax.fori_loop(0, 2000 // 16, zd, 0)
                for j in range(n // 2000):
                    pltpu.sync_copy(zerod_v, deg_sh.at[pl.ds(j * 2000, 2000)])

        plsc.subcore_barrier()

        def load_sidx(j, q):
            off = pl.multiple_of(ebase + j * C, 8)
            pltpu.async_copy(src_hbm.at[pl.ds(off, C)], sidx[q], isem[q])

        def wait_sidx(q):
            pltpu.make_async_copy(src_hbm.at[pl.ds(0, C)], sidx[q],
                                  isem[q]).wait()

        def load_didx(j, q):
            off = pl.multiple_of(ebase + j * C, 8)
            pltpu.async_copy(dst_hbm.at[pl.ds(off, C)], didx[q], jsem[q])

        def wait_didx(q):
            pltpu.make_async_copy(dst_hbm.at[pl.ds(0, C)], didx[q],
                                  jsem[q]).wait()

        def start_gather(q):
            pltpu.async_copy(x_hbm.at[sidx[q]], rows[q], gsem[q])

        def wait_gather(q):
            pltpu.make_async_copy(x_hbm.at[pl.ds(0, C), :], rows[q],
                                  gsem[q]).wait()

        def issue_scatters(q):
            pltpu.async_copy(rows[q], agg_sh.at[didx[q]], ssem[q], add=True)
            if with_deg:
                pltpu.async_copy(ones_v, deg_sh.at[didx[q]], ssem[q],
                                 add=True)

        def drain_scatters(q):
            pltpu.make_async_copy(rows[q], agg_sh.at[pl.ds(0, C), :],
                                  ssem[q]).wait()
            if with_deg:
                pltpu.make_async_copy(ones_v, deg_sh.at[pl.ds(0, C)],
                                      ssem[q]).wait()

        load_sidx(0, 0)
        load_didx(0, 0)
        wait_sidx(0)
        start_gather(0)
        load_sidx(1, 1)
        load_sidx(2, 2)
        wait_sidx(1)
        start_gather(1)

        def eb(jj, carry):
            for u in range(3):
                j = jj * 3 + u
                q = u % 3
                q1 = (u + 1) % 3
                q2 = (u + 2) % 3

                @pl.when(j > 0)
                def _():
                    drain_scatters(q2)

                @pl.when(j + 2 < full)
                def _():
                    wait_sidx(q2)
                    start_gather(q2)

                @pl.when(j + 1 < full)
                def _():
                    load_didx(j + 1, q1)
                wait_gather(q)

                @pl.when(j + 3 < full)
                def _():
                    load_sidx(j + 3, q)
                wait_didx(q)
                issue_scatters(q)
            return carry
        lax.fori_loop(0, full // 3, eb, 0)
        drain_scatters((full - 1) % 3)

        plsc.subcore_barrier()

        pltpu.sync_copy(agg_sh.at[pl.ds(tbase, tr), :],
                        agg_hbm.at[cid, pl.ds(tbase, tr), :])
        if rtail:
            @pl.when(sid == NS - 1)
            def _():
                pltpu.sync_copy(agg_sh.at[pl.ds(rtail_off, rtail), :],
                                agg_hbm.at[cid, pl.ds(rtail_off, rtail), :])
        if with_deg:
            @pl.when(sid == 0)
            def _():
                pltpu.sync_copy(deg_sh, deg_hbm.at[cid])

    return pl.kernel(body, out_type=out_type, mesh=mesh,
                     scratch_types=scratch)


def _make_tc_dense(n, d, r):
    assert n % r == 0

    def body(x_ref, agg_ref, deg_ref, ws_ref, wn_ref, b_ref, o_ref):
        agg = agg_ref[0] + agg_ref[1]
        deg = deg_ref[0] + deg_ref[1]
        rdeg = 1.0 / jnp.maximum(deg, 1.0)
        acc = jnp.dot(x_ref[...], ws_ref[...],
                      preferred_element_type=jnp.float32)
        acc = acc + jnp.dot(agg * rdeg, wn_ref[...],
                            preferred_element_type=jnp.float32)
        o_ref[...] = jnp.maximum(acc + b_ref[...], 0.0)

    return pl.pallas_call(
        body,
        grid=(n // r,),
        in_specs=[
            pl.BlockSpec((r, d), lambda i: (i, 0)),
            pl.BlockSpec((NC, r, d), lambda i: (0, i, 0)),
            pl.BlockSpec((NC, r, 1), lambda i: (0, i, 0)),
            pl.BlockSpec((d, d), lambda i: (0, 0)),
            pl.BlockSpec((d, d), lambda i: (0, 0)),
            pl.BlockSpec((1, d), lambda i: (0, 0)),
        ],
        out_specs=pl.BlockSpec((r, d), lambda i: (i, 0)),
        out_shape=jax.ShapeDtypeStruct((n, d), jnp.float32),
    )


def kernel(x, edge_index, W_self1, W_neigh1, b1, W_self2, W_neigh2, b2):
    n, d = x.shape
    e = edge_index.shape[1]
    assert e % NW == 0
    epw0 = e // NW
    epw = -(-epw0 // (3 * C)) * (3 * C)
    pad = epw - epw0

    src = edge_index[0].reshape(NW, epw0)
    dst = edge_index[1].reshape(NW, epw0)
    if pad:
        fill = jnp.arange(NW * pad, dtype=jnp.int32).reshape(NW, pad)
        src = jnp.concatenate([src, (fill * 131) % n], axis=1)
        dst = jnp.concatenate([dst, n + (fill % ND)], axis=1)
    src = src.reshape(-1)
    dst = dst.reshape(-1)

    sc_agg_deg = _make_sc_agg(n, d, epw, with_deg=True)
    sc_agg = _make_sc_agg(n, d, epw, with_deg=False)
    tc_dense = _make_tc_dense(n, d, 1000)

    aggp1, degp = sc_agg_deg(x, src, dst)
    deg_col = degp[:, :n].reshape(NC, n, 1)
    h = tc_dense(x, aggp1, deg_col, W_self1, W_neigh1, b1.reshape(1, d))
    aggp2 = sc_agg(h, src, dst)
    return tc_dense(h, aggp2, deg_col, W_self2, W_neigh2, b2.reshape(1, d))


# speedup vs baseline: 15.2522x; 1.0264x over previous
import jax
import jax.numpy as jnp
from jax import lax
from jax.experimental import pallas as pl
from jax.experimental.pallas import tpu as pltpu
from jax.experimental.pallas import tpu_sc as plsc

NC = 2
NS = 16
NW = NC * NS
C = 112
ND = 8


def _make_sc_agg(n, d, epw, with_deg):
    assert epw % C == 0
    full = epw // C
    assert full % 3 == 0
    tr = (n // NS) & ~7
    rtail_off = tr * NS
    rtail = n - rtail_off
    zrows = 24
    assert tr % zrows == 0 and 0 <= rtail <= zrows and rtail % 8 == 0

    if with_deg:
        out_type = (jax.ShapeDtypeStruct((NC, n, d), jnp.float32),
                    jax.ShapeDtypeStruct((NC, n + ND), jnp.float32))
    else:
        out_type = jax.ShapeDtypeStruct((NC, n, d), jnp.float32)

    scratch = [
        pltpu.VMEM_SHARED((n + ND, d), jnp.float32),
        [pltpu.VMEM((C,), jnp.int32)] * 3,
        [pltpu.VMEM((C,), jnp.int32)] * 3,
        [pltpu.VMEM((C, d), jnp.float32)] * 3,
        pltpu.VMEM((zrows, d), jnp.float32),
        [pltpu.SemaphoreType.DMA] * 3,
        [pltpu.SemaphoreType.DMA] * 3,
        [pltpu.SemaphoreType.DMA] * 3,
        [pltpu.SemaphoreType.DMA] * 3,
        pltpu.SemaphoreType.DMA,
    ]
    if with_deg:
        scratch += [
            pltpu.VMEM_SHARED((n + ND,), jnp.float32),
            pltpu.VMEM((C,), jnp.float32),
            pltpu.VMEM((2000,), jnp.float32),
        ]

    mesh = plsc.VectorSubcoreMesh(core_axis_name="c", subcore_axis_name="s")

    def body(x_hbm, src_hbm, dst_hbm, *refs):
        if with_deg:
            (agg_hbm, deg_hbm, agg_sh, sidx, didx, rows, zero_v, gsem, isem,
             jsem, ssem, zsem, deg_sh, ones_v, zerod_v) = refs
        else:
            (agg_hbm, agg_sh, sidx, didx, rows, zero_v, gsem, isem, jsem,
             ssem, zsem) = refs
        cid = lax.axis_index("c")
        sid = lax.axis_index("s")
        wid = sid * NC + cid
        tbase = pl.multiple_of(sid * tr, 8)
        ebase = pl.multiple_of(wid * epw, 8)

        def zb(i, carry):
            zero_v[i // (d // 16), pl.ds((i % (d // 16)) * 16, 16)] = (
                jnp.zeros((16,), jnp.float32))
            return carry
        lax.fori_loop(0, zrows * d // 16, zb, 0)
        for j in range(tr // zrows):
            pltpu.async_copy(
                zero_v,
                agg_sh.at[pl.ds(pl.multiple_of(tbase + j * zrows, 8),
                                zrows), :], zsem)
        if rtail:
            @pl.when(sid == NS - 1)
            def _():
                pltpu.async_copy(zero_v.at[pl.ds(0, rtail), :],
                                 agg_sh.at[pl.ds(rtail_off, rtail), :], zsem)

        if with_deg:
            for k in range(C // 16):
                ones_v[pl.ds(k * 16, 16)] = jnp.full((16,), 1.0, jnp.float32)

            @pl.when(sid == 0)
            def _():
                def zd(i, carry):
                    zerod_v[pl.ds(i * 16, 16)] = jnp.zeros((16,), jnp.float32)
                    return carry
                lax.fori_loop(0, 2000 // 16, zd, 0)
                for j in range(n // 2000):
                    pltpu.async_copy(zerod_v, deg_sh.at[pl.ds(j * 2000, 2000)],
                                     zsem)

        def load_sidx(j, q):
            off = pl.multiple_of(ebase + j * C, 8)
            pltpu.async_copy(src_hbm.at[pl.ds(off, C)], sidx[q], isem[q])

        def wait_sidx(q):
            pltpu.make_async_copy(src_hbm.at[pl.ds(0, C)], sidx[q],
                                  isem[q]).wait()

        def load_didx(j, q):
            off = pl.multiple_of(ebase + j * C, 8)
            pltpu.async_copy(dst_hbm.at[pl.ds(off, C)], didx[q], jsem[q])

        def wait_didx(q):
            pltpu.make_async_copy(dst_hbm.at[pl.ds(0, C)], didx[q],
                                  jsem[q]).wait()

        def start_gather(q):
            pltpu.async_copy(x_hbm.at[sidx[q]], rows[q], gsem[q])

        def wait_gather(q):
            pltpu.make_async_copy(x_hbm.at[pl.ds(0, C), :], rows[q],
                                  gsem[q]).wait()

        def issue_scatters(q):
            pltpu.async_copy(rows[q], agg_sh.at[didx[q]], ssem[q], add=True)
            if with_deg:
                pltpu.async_copy(ones_v, deg_sh.at[didx[q]], ssem[q],
                                 add=True)

        def drain_scatters(q):
            pltpu.make_async_copy(rows[q], agg_sh.at[pl.ds(0, C), :],
                                  ssem[q]).wait()
            if with_deg:
                pltpu.make_async_copy(ones_v, deg_sh.at[pl.ds(0, C)],
                                      ssem[q]).wait()

        load_sidx(0, 0)
        load_didx(0, 0)
        load_sidx(1, 1)
        load_sidx(2, 2)
        wait_sidx(0)
        start_gather(0)
        wait_sidx(1)
        start_gather(1)

        for j in range(tr // zrows):
            pltpu.make_async_copy(zero_v, agg_sh.at[pl.ds(0, zrows), :],
                                  zsem).wait()
        if rtail:
            @pl.when(sid == NS - 1)
            def _():
                pltpu.make_async_copy(zero_v.at[pl.ds(0, rtail), :],
                                      agg_sh.at[pl.ds(0, rtail), :],
                                      zsem).wait()
        if with_deg:
            @pl.when(sid == 0)
            def _():
                for j in range(n // 2000):
                    pltpu.make_async_copy(zerod_v,
                                          deg_sh.at[pl.ds(0, 2000)],
                                          zsem).wait()

        plsc.subcore_barrier()

        def eb(jj, carry):
            for u in range(3):
                j = jj * 3 + u
                q = u % 3
                q1 = (u + 1) % 3
                q2 = (u + 2) % 3

                @pl.when(j > 0)
                def _():
                    drain_scatters(q2)

                @pl.when(j + 2 < full)
                def _():
                    wait_sidx(q2)
                    start_gather(q2)

                @pl.when(j + 1 < full)
                def _():
                    load_didx(j + 1, q1)
                wait_gather(q)

                @pl.when(j + 3 < full)
                def _():
                    load_sidx(j + 3, q)
                wait_didx(q)
                issue_scatters(q)
            return carry
        lax.fori_loop(0, full // 3, eb, 0)
        drain_scatters((full - 1) % 3)

        plsc.subcore_barrier()

        pltpu.sync_copy(agg_sh.at[pl.ds(tbase, tr), :],
                        agg_hbm.at[cid, pl.ds(tbase, tr), :])
        if rtail:
            @pl.when(sid == NS - 1)
            def _():
                pltpu.sync_copy(agg_sh.at[pl.ds(rtail_off, rtail), :],
                                agg_hbm.at[cid, pl.ds(rtail_off, rtail), :])
        if with_deg:
            @pl.when(sid == 0)
            def _():
                pltpu.sync_copy(deg_sh, deg_hbm.at[cid])

    return pl.kernel(body, out_type=out_type, mesh=mesh,
                     scratch_types=scratch)


def _make_tc_dense(n, d, r):
    assert n % r == 0

    def body(x_ref, agg_ref, deg_ref, ws_ref, wn_ref, b_ref, o_ref):
        agg = agg_ref[0] + agg_ref[1]
        deg = deg_ref[0] + deg_ref[1]
        rdeg = 1.0 / jnp.maximum(deg, 1.0)
        acc = jnp.dot(x_ref[...], ws_ref[...],
                      preferred_element_type=jnp.float32)
        acc = acc + jnp.dot(agg * rdeg, wn_ref[...],
                            preferred_element_type=jnp.float32)
        o_ref[...] = jnp.maximum(acc + b_ref[...], 0.0)

    return pl.pallas_call(
        body,
        grid=(n // r,),
        in_specs=[
            pl.BlockSpec((r, d), lambda i: (i, 0)),
            pl.BlockSpec((NC, r, d), lambda i: (0, i, 0)),
            pl.BlockSpec((NC, r, 1), lambda i: (0, i, 0)),
            pl.BlockSpec((d, d), lambda i: (0, 0)),
            pl.BlockSpec((d, d), lambda i: (0, 0)),
            pl.BlockSpec((1, d), lambda i: (0, 0)),
        ],
        out_specs=pl.BlockSpec((r, d), lambda i: (i, 0)),
        out_shape=jax.ShapeDtypeStruct((n, d), jnp.float32),
    )


def kernel(x, edge_index, W_self1, W_neigh1, b1, W_self2, W_neigh2, b2):
    n, d = x.shape
    e = edge_index.shape[1]
    assert e % NW == 0
    epw0 = e // NW
    epw = -(-epw0 // (3 * C)) * (3 * C)
    pad = epw - epw0

    src = edge_index[0].reshape(NW, epw0)
    dst = edge_index[1].reshape(NW, epw0)
    if pad:
        fill = jnp.arange(NW * pad, dtype=jnp.int32).reshape(NW, pad)
        src = jnp.concatenate([src, (fill * 131) % n], axis=1)
        dst = jnp.concatenate([dst, n + (fill % ND)], axis=1)
    src = src.reshape(-1)
    dst = dst.reshape(-1)

    sc_agg_deg = _make_sc_agg(n, d, epw, with_deg=True)
    sc_agg = _make_sc_agg(n, d, epw, with_deg=False)
    tc_dense = _make_tc_dense(n, d, 1000)

    aggp1, degp = sc_agg_deg(x, src, dst)
    deg_col = degp[:, :n].reshape(NC, n, 1)
    h = tc_dense(x, aggp1, deg_col, W_self1, W_neigh1, b1.reshape(1, d))
    aggp2 = sc_agg(h, src, dst)
    return tc_dense(h, aggp2, deg_col, W_self2, W_neigh2, b2.reshape(1, d))
